# Initial kernel scaffold; baseline (speedup 1.0000x reference)
import jax
import jax.numpy as jnp
from jax.experimental import pallas as pl


def kernel(in_map):
    raise NotImplementedError("write your pallas kernel here")


# speedup vs baseline: 8.0393x; 8.0393x over previous
import numpy as np
import jax
import jax.numpy as jnp
from jax import lax
from jax.experimental import pallas as pl
from jax.experimental.pallas import tpu as pltpu
from jax.experimental.pallas import tpu_sc as plsc

K = 1024
H = 512
W = 512
R = 64
N = H * W
CAPB = 2048
CAP = CAPB - 16
NC, NS, L = 2, 16, 16
NSUB = NC * NS
ROWS_PER_SUB = R // NSUB
CH = 8192
NCH = N // CH

_I32_MIN = np.int32(-(2 ** 31))
_I32_M7F = np.int32(0x7FFFFFFF)


def _take16(x, idx):
    return lax.gather(
        x, idx[:, None],
        dimension_numbers=lax.GatherDimensionNumbers(
            offset_dims=(), collapsed_slice_dims=(0,), start_index_map=(0,)),
        slice_sizes=(1,),
        mode=lax.GatherScatterMode.PROMISE_IN_BOUNDS)


def _threshold_kernel(x_ref, s_ref, sth_ref, maxes_ref):
    r = pl.program_id(0)
    x = x_ref[0, 0]
    b = lax.bitcast_convert_type(x, jnp.int32)
    s = b ^ ((b >> 31) & _I32_M7F)
    s_ref[0] = s
    maxes_ref[r] = jnp.max(s.reshape(128, 4, W), axis=0)

    @pl.when(r == R - 1)
    def _():
        sm = maxes_ref[...]
        wbase = jnp.zeros((R, 1, 1), jnp.int32)
        for bit in range(31, -1, -1):
            cw = wbase | np.int32(np.uint32(1 << bit))
            cs = cw ^ _I32_MIN
            cnt = jnp.sum((sm >= cs).astype(jnp.int32), axis=(1, 2),
                          keepdims=True)
            wbase = jnp.where(cnt >= K, cw, wbase)
        sstar = wbase ^ _I32_MIN
        sth_ref[...] = jnp.broadcast_to((sstar >> 1).reshape(R, 1), (R, 128))


def _compute_thresholds(in_map):
    return pl.pallas_call(
        _threshold_kernel,
        grid=(R,),
        in_specs=[pl.BlockSpec((1, 1, H, W), lambda r: (r, 0, 0, 0))],
        out_specs=(
            pl.BlockSpec((1, H, W), lambda r: (r, 0, 0)),
            pl.BlockSpec((R, 128), lambda r: (0, 0)),
        ),
        out_shape=(
            jax.ShapeDtypeStruct((R, H, W), jnp.int32),
            jax.ShapeDtypeStruct((R, 128), jnp.int32),
        ),
        scratch_shapes=[pltpu.VMEM((R, 4, W), jnp.int32)],
    )(in_map)


def _sc_compact_kernel(s_hbm, sth_hbm, sv_hbm, ci_hbm, cnt_hbm,
                       buf, thr_v, sv_v, ci_v, cnt_v):
    cid = lax.axis_index("c")
    sid = lax.axis_index("s")
    wid = sid * NC + cid
    lane = lax.iota(jnp.int32, L)

    for rr in range(ROWS_PER_SUB):
        r = wid * ROWS_PER_SUB + rr
        pltpu.sync_copy(sth_hbm.at[pl.ds(r * 128, L)], thr_v)

        def chunk_body(c, carry, r=r):
            cntv, idxv = carry
            pltpu.sync_copy(s_hbm.at[pl.ds(r * N + c * CH, CH)], buf)

            def step(j, carry):
                cnt, idxv = carry
                sv = buf[pl.ds(j * L, L)]
                sth = thr_v[...]
                mi = 1 + (((sv >> 1) - sth) >> 31)
                cs = mi
                for d in (1, 2, 4, 8):
                    src = jnp.maximum(lane - d, 0)
                    gm = 1 + ((lane - d) >> 31)
                    cs = cs + _take16(cs, src) * gm
                p = lane * 0
                for d in (8, 4, 2, 1):
                    csp = _take16(cs, p + (d - 1))
                    ge = 1 + ((csp - (lane + 1)) >> 31)
                    p = p + (1 - ge) * d
                off = jnp.minimum(cnt, CAP)
                sv_v[pl.ds(off, L)] = _take16(sv, p)
                ci_v[pl.ds(off, L)] = _take16(idxv, p)
                tot = cs[L - 1]
                return cnt + tot, idxv + L

            return lax.fori_loop(0, CH // L, step, (cntv, idxv))

        cnt, _ = lax.fori_loop(0, NCH, chunk_body, (jnp.int32(0), lane))
        cnt_v[...] = lane * 0 + cnt
        pltpu.sync_copy(sv_v.at[pl.ds(0, CAPB)], sv_hbm.at[pl.ds(r * CAPB, CAPB)])
        pltpu.sync_copy(ci_v.at[pl.ds(0, CAPB)], ci_hbm.at[pl.ds(r * CAPB, CAPB)])
        pltpu.sync_copy(cnt_v, cnt_hbm.at[pl.ds(r * L, L)])


def _sc_compact(s1d, sth1d):
    mesh = plsc.VectorSubcoreMesh(core_axis_name="c", subcore_axis_name="s",
                                  num_cores=NC, num_subcores=NS)
    fn = pl.kernel(
        _sc_compact_kernel,
        out_type=(
            jax.ShapeDtypeStruct((R * CAPB,), jnp.int32),
            jax.ShapeDtypeStruct((R * CAPB,), jnp.int32),
            jax.ShapeDtypeStruct((R * L,), jnp.int32),
        ),
        mesh=mesh,
        scratch_types=[
            pltpu.VMEM((CH,), jnp.int32),
            pltpu.VMEM((L,), jnp.int32),
            pltpu.VMEM((CAPB + L,), jnp.int32),
            pltpu.VMEM((CAPB + L,), jnp.int32),
            pltpu.VMEM((L,), jnp.int32),
        ],
    )
    return fn(s1d, sth1d)


def _sort_kernel(sv_ref, ci_ref, cnt_ref, ids_ref, x_ref, y_ref):
    s = sv_ref[...]
    ix = ci_ref[...]
    cnt = jnp.minimum(cnt_ref[:, 0:1], CAPB)
    pos = lax.broadcasted_iota(jnp.int32, (R, CAPB), 1)
    valid = pos < cnt
    s = jnp.where(valid, s, _I32_MIN)
    ix = jnp.where(valid, ix, jnp.int32(2 ** 30))

    k = 2
    while k <= CAPB:
        j = k // 2
        while j >= 1:
            low = (pos & j) == 0
            ps = jnp.where(low, jnp.roll(s, -j, axis=1),
                           jnp.roll(s, j, axis=1))
            pix = jnp.where(low, jnp.roll(ix, -j, axis=1),
                            jnp.roll(ix, j, axis=1))
            g = (s > ps) | ((s == ps) & (ix < pix))
            tw = low == ((pos & k) == 0)
            take_self = tw == g
            s = jnp.where(take_self, s, ps)
            ix = jnp.where(take_self, ix, pix)
            j //= 2
        k *= 2

    ids = ix[:, :K]
    ids_ref[...] = ids
    x_ref[...] = (ids & (W - 1)).astype(jnp.float32) * jnp.float32(1.0 / W)
    y_ref[...] = (ids >> 9).astype(jnp.float32) * jnp.float32(1.0 / H)


def _sort_candidates(sv, ci, cnt):
    return pl.pallas_call(
        _sort_kernel,
        out_shape=(
            jax.ShapeDtypeStruct((R, K), jnp.int32),
            jax.ShapeDtypeStruct((R, K), jnp.float32),
            jax.ShapeDtypeStruct((R, K), jnp.float32),
        ),
    )(sv, ci, cnt)


def kernel(in_map):
    smap, sth = _compute_thresholds(in_map)
    sv1d, ci1d, cnt1d = _sc_compact(smap.reshape(R * N), sth.reshape(R * 128))
    ids, xs, ys = _sort_candidates(sv1d.reshape(R, CAPB),
                                   ci1d.reshape(R, CAPB),
                                   cnt1d.reshape(R, L))
    pts_xy = jnp.stack([xs, ys], axis=2)
    return ids, pts_xy


# speedup vs baseline: 8.1628x; 1.0154x over previous
import numpy as np
import jax
import jax.numpy as jnp
from jax import lax
from jax.experimental import pallas as pl
from jax.experimental.pallas import tpu as pltpu
from jax.experimental.pallas import tpu_sc as plsc

K = 1024
H = 512
W = 512
R = 64
N = H * W
CAPB = 2048
CAP = CAPB - 16
NC, NS, L = 2, 16, 16
NSUB = NC * NS
ROWS_PER_SUB = R // NSUB
CH = 16384
NCH = N // CH

_I32_MIN = np.int32(-(2 ** 31))
_I32_M7F = np.int32(0x7FFFFFFF)


def _take16(x, idx):
    return lax.gather(
        x, idx[:, None],
        dimension_numbers=lax.GatherDimensionNumbers(
            offset_dims=(), collapsed_slice_dims=(0,), start_index_map=(0,)),
        slice_sizes=(1,),
        mode=lax.GatherScatterMode.PROMISE_IN_BOUNDS)


def _threshold_kernel(x_ref, s_ref, sth_ref, maxes_ref):
    r = pl.program_id(0)
    x = x_ref[0, 0]
    b = lax.bitcast_convert_type(x, jnp.int32)
    s = b ^ ((b >> 31) & _I32_M7F)
    s_ref[0] = s
    maxes_ref[r] = jnp.max(s.reshape(128, 4, W), axis=0)

    @pl.when(r == R - 1)
    def _():
        sm = maxes_ref[...]
        wbase = jnp.zeros((R, 1, 1), jnp.int32)
        for bit in range(31, -1, -1):
            cw = wbase | np.int32(np.uint32(1 << bit))
            cs = cw ^ _I32_MIN
            cnt = jnp.sum((sm >= cs).astype(jnp.int32), axis=(1, 2),
                          keepdims=True)
            wbase = jnp.where(cnt >= K, cw, wbase)
        sstar = wbase ^ _I32_MIN
        sth_ref[...] = jnp.broadcast_to((sstar >> 1).reshape(R, 1), (R, 128))


def _compute_thresholds(in_map):
    return pl.pallas_call(
        _threshold_kernel,
        grid=(R,),
        in_specs=[pl.BlockSpec((1, 1, H, W), lambda r: (r, 0, 0, 0))],
        out_specs=(
            pl.BlockSpec((1, H, W), lambda r: (r, 0, 0)),
            pl.BlockSpec((R, 128), lambda r: (0, 0)),
        ),
        out_shape=(
            jax.ShapeDtypeStruct((R, H, W), jnp.int32),
            jax.ShapeDtypeStruct((R, 128), jnp.int32),
        ),
        scratch_shapes=[pltpu.VMEM((R, 4, W), jnp.int32)],
    )(in_map)


def _sc_compact_kernel(s_hbm, sth_hbm, sv_hbm, ci_hbm, cnt_hbm,
                       buf, thr_v, sv_v, ci_v, cnt_v):
    cid = lax.axis_index("c")
    sid = lax.axis_index("s")
    wid = sid * NC + cid
    lane = lax.iota(jnp.int32, L)

    for rr in range(ROWS_PER_SUB):
        r = wid * ROWS_PER_SUB + rr
        pltpu.sync_copy(sth_hbm.at[pl.ds(r * 128, L)], thr_v)

        def chunk_body(c, carry, r=r):
            cntv, idxv = carry
            pltpu.sync_copy(s_hbm.at[pl.ds(r * N + c * CH, CH)], buf)

            def step(j, carry):
                cnt, idxv = carry
                sv = buf[pl.ds(j * L, L)]
                sth = thr_v[...]
                mi = 1 + (((sv >> 1) - sth) >> 31)
                cs = mi
                for d in (1, 2, 4, 8):
                    src = jnp.maximum(lane - d, 0)
                    gm = 1 + ((lane - d) >> 31)
                    cs = cs + _take16(cs, src) * gm
                p = lane * 0
                for d in (8, 4, 2, 1):
                    csp = _take16(cs, p + (d - 1))
                    ge = 1 + ((csp - (lane + 1)) >> 31)
                    p = p + (1 - ge) * d
                off = jnp.minimum(cnt, CAP)
                sv_v[pl.ds(off, L)] = _take16(sv, p)
                ci_v[pl.ds(off, L)] = _take16(idxv, p)
                tot = cs[L - 1]
                return cnt + tot, idxv + L

            def step2(j, carry):
                return step(2 * j + 1, step(2 * j, carry))

            return lax.fori_loop(0, CH // L // 2, step2, (cntv, idxv))

        cnt, _ = lax.fori_loop(0, NCH, chunk_body, (jnp.int32(0), lane))
        cnt_v[...] = lane * 0 + cnt
        pltpu.sync_copy(sv_v.at[pl.ds(0, CAPB)], sv_hbm.at[pl.ds(r * CAPB, CAPB)])
        pltpu.sync_copy(ci_v.at[pl.ds(0, CAPB)], ci_hbm.at[pl.ds(r * CAPB, CAPB)])
        pltpu.sync_copy(cnt_v, cnt_hbm.at[pl.ds(r * L, L)])


def _sc_compact(s1d, sth1d):
    mesh = plsc.VectorSubcoreMesh(core_axis_name="c", subcore_axis_name="s",
                                  num_cores=NC, num_subcores=NS)
    fn = pl.kernel(
        _sc_compact_kernel,
        out_type=(
            jax.ShapeDtypeStruct((R * CAPB,), jnp.int32),
            jax.ShapeDtypeStruct((R * CAPB,), jnp.int32),
            jax.ShapeDtypeStruct((R * L,), jnp.int32),
        ),
        mesh=mesh,
        scratch_types=[
            pltpu.VMEM((CH,), jnp.int32),
            pltpu.VMEM((L,), jnp.int32),
            pltpu.VMEM((CAPB + L,), jnp.int32),
            pltpu.VMEM((CAPB + L,), jnp.int32),
            pltpu.VMEM((L,), jnp.int32),
        ],
    )
    return fn(s1d, sth1d)


def _sort_kernel(sv_ref, ci_ref, cnt_ref, ids_ref, x_ref, y_ref):
    s = sv_ref[...]
    ix = ci_ref[...]
    cnt = jnp.minimum(cnt_ref[:, 0:1], CAPB)
    pos = lax.broadcasted_iota(jnp.int32, (R, CAPB), 1)
    valid = pos < cnt
    s = jnp.where(valid, s, _I32_MIN)
    ix = jnp.where(valid, ix, jnp.int32(2 ** 30))

    k = 2
    while k <= CAPB:
        j = k // 2
        while j >= 1:
            low = (pos & j) == 0
            ps = jnp.where(low, jnp.roll(s, -j, axis=1),
                           jnp.roll(s, j, axis=1))
            pix = jnp.where(low, jnp.roll(ix, -j, axis=1),
                            jnp.roll(ix, j, axis=1))
            g = (s > ps) | ((s == ps) & (ix < pix))
            tw = low == ((pos & k) == 0)
            take_self = tw == g
            s = jnp.where(take_self, s, ps)
            ix = jnp.where(take_self, ix, pix)
            j //= 2
        k *= 2

    ids = ix[:, :K]
    ids_ref[...] = ids
    x_ref[...] = (ids & (W - 1)).astype(jnp.float32) * jnp.float32(1.0 / W)
    y_ref[...] = (ids >> 9).astype(jnp.float32) * jnp.float32(1.0 / H)


def _sort_candidates(sv, ci, cnt):
    return pl.pallas_call(
        _sort_kernel,
        out_shape=(
            jax.ShapeDtypeStruct((R, K), jnp.int32),
            jax.ShapeDtypeStruct((R, K), jnp.float32),
            jax.ShapeDtypeStruct((R, K), jnp.float32),
        ),
    )(sv, ci, cnt)


def kernel(in_map):
    smap, sth = _compute_thresholds(in_map)
    sv1d, ci1d, cnt1d = _sc_compact(smap.reshape(R * N), sth.reshape(R * 128))
    ids, xs, ys = _sort_candidates(sv1d.reshape(R, CAPB),
                                   ci1d.reshape(R, CAPB),
                                   cnt1d.reshape(R, L))
    pts_xy = jnp.stack([xs, ys], axis=2)
    return ids, pts_xy
